# baseline (speedup 1.0000x reference)
import jax
import jax.numpy as jnp
from jax.experimental import pallas as pl
from jax.experimental.pallas import tpu as pltpu

B, L, D = 16, 1024, 256
W = L // 2
K = 2048


def _cra_kernel(ps_ref, x2_ref, c2_ref, cb_ref, w_ref, b_ref,
                idx_ref, emb_ref, loss_ref, s_ref, acc_ref):
    step = pl.program_id(0)

    @pl.when(step == 0)
    def _init():
        cb = cb_ref[...]
        proj = jax.lax.dot_general(
            cb, w_ref[...], (((1,), (0,)), ((), ())),
            precision=jax.lax.Precision.DEFAULT,
            preferred_element_type=jnp.float32) + b_ref[...]
        r = proj - cb
        s_ref[0, :] = jnp.sum(r * r, axis=1)
        acc_ref[0, 0] = 0.0

    wm = ps_ref[0] * 0.5
    cb = cb_ref[...]
    e = jax.lax.dot_general(
        wm, cb, (((1,), (1,)), ((), ())),
        precision=jax.lax.Precision.DEFAULT,
        preferred_element_type=jnp.float32)
    x2 = x2_ref[0, 0, :][:, None]
    d2 = (x2 - 2.0 * e) + c2_ref[0][None, :]

    lanes = jax.lax.broadcasted_iota(jnp.int32, (W, K), 1)
    mn = jnp.min(d2, axis=1, keepdims=True)
    idx = jnp.min(jnp.where(d2 == mn, lanes, K), axis=1).astype(jnp.int32)
    idx_ref[0, 0, :] = idx

    onehot = (lanes == idx[:, None]).astype(jnp.float32)
    emb_ref[0] = jax.lax.dot_general(
        onehot, cb, (((1,), (0,)), ((), ())),
        precision=jax.lax.Precision.HIGHEST,
        preferred_element_type=jnp.float32)

    acc_ref[0, 0] += jnp.sum(onehot * s_ref[0, :][None, :])

    @pl.when(step == B - 1)
    def _fin():
        loss_ref[0, 0] = acc_ref[0, 0] * (1.0 / (B * W * D))


@jax.jit
def kernel(char_tokens, char_embeddings, word_codebook, W_proj, b_proj):
    del char_tokens
    ps = (char_embeddings[:, 0::2, :] + char_embeddings[:, 1::2, :])
    ps = jax.lax.optimization_barrier(ps)
    wm_o = ps * 0.5
    x2 = jnp.sum(wm_o * wm_o, axis=-1)
    c2 = jnp.sum(word_codebook * word_codebook, axis=-1)

    b2 = b_proj.reshape(1, D)
    idx, emb, loss = pl.pallas_call(
        _cra_kernel,
        grid=(B,),
        in_specs=[
            pl.BlockSpec((1, W, D), lambda i: (i, 0, 0)),
            pl.BlockSpec((1, 1, W), lambda i: (i, 0, 0)),
            pl.BlockSpec((1, K), lambda i: (0, 0)),
            pl.BlockSpec((K, D), lambda i: (0, 0)),
            pl.BlockSpec((D, D), lambda i: (0, 0)),
            pl.BlockSpec((1, D), lambda i: (0, 0)),
        ],
        out_specs=[
            pl.BlockSpec((1, 1, W), lambda i: (i, 0, 0)),
            pl.BlockSpec((1, W, D), lambda i: (i, 0, 0)),
            pl.BlockSpec(memory_space=pltpu.SMEM),
        ],
        out_shape=[
            jax.ShapeDtypeStruct((B, 1, W), jnp.int32),
            jax.ShapeDtypeStruct((B, W, D), jnp.float32),
            jax.ShapeDtypeStruct((1, 1), jnp.float32),
        ],
        scratch_shapes=[
            pltpu.VMEM((1, K), jnp.float32),
            pltpu.SMEM((1, 1), jnp.float32),
        ],
        compiler_params=pltpu.CompilerParams(
            dimension_semantics=("arbitrary",)),
    )(ps, x2.reshape(B, 1, W), c2.reshape(1, K), word_codebook, W_proj, b2)
    return (idx.reshape(B, W), emb, loss.reshape(()))


# speedup vs baseline: 1.3851x; 1.3851x over previous
import jax
import jax.numpy as jnp
from jax.experimental import pallas as pl
from jax.experimental.pallas import tpu as pltpu

B, L, D = 16, 1024, 256
W = L // 2
K = 2048


def _rowsum_ref_order(sq):
    m = sq[:, :128] + sq[:, 128:]
    r = m[:, 0:8]
    for k in range(1, 16):
        r = r + m[:, 8 * k:8 * k + 8]
    u = r[:, 0:4] + r[:, 4:8]
    v = u[:, 0:2] + u[:, 2:4]
    return v[:, 0:1] + v[:, 1:2]


def _cra_kernel(ce_ref, c2_ref, cb_ref, w_ref, b_ref,
                idx_ref, emb_ref, loss_ref, s_ref, acc_ref):
    step = pl.program_id(0)

    @pl.when(step == 0)
    def _init():
        cb = cb_ref[...]
        proj = jax.lax.dot_general(
            cb, w_ref[...], (((1,), (0,)), ((), ())),
            precision=jax.lax.Precision.DEFAULT,
            preferred_element_type=jnp.float32) + b_ref[...]
        r = proj - cb
        s_ref[0, :] = jnp.sum(r * r, axis=1)
        acc_ref[0, 0] = 0.0

    ce = ce_ref[0]
    wm = (ce[:, :D] + ce[:, D:]) * 0.5

    cb = cb_ref[...]
    e = jax.lax.dot_general(
        wm, cb, (((1,), (1,)), ((), ())),
        precision=jax.lax.Precision.DEFAULT,
        preferred_element_type=jnp.float32)
    x2 = _rowsum_ref_order(wm * wm)
    d2 = (x2 - 2.0 * e) + c2_ref[0][None, :]

    lanes = jax.lax.broadcasted_iota(jnp.int32, (W, K), 1)
    mn = jnp.min(d2, axis=1, keepdims=True)
    idx = jnp.min(jnp.where(d2 == mn, lanes, K), axis=1).astype(jnp.int32)
    idx_ref[0, 0, :] = idx

    onehot = (lanes == idx[:, None]).astype(jnp.float32)
    emb_ref[0] = jax.lax.dot_general(
        onehot, cb, (((1,), (0,)), ((), ())),
        precision=jax.lax.Precision.HIGHEST,
        preferred_element_type=jnp.float32)

    acc_ref[0, 0] += jnp.sum(onehot * s_ref[0, :][None, :])

    @pl.when(step == B - 1)
    def _fin():
        loss_ref[0, 0] = acc_ref[0, 0] * (1.0 / (B * W * D))


@jax.jit
def kernel(char_tokens, char_embeddings, word_codebook, W_proj, b_proj):
    del char_tokens
    c2 = jnp.sum(word_codebook * word_codebook, axis=-1)
    b2 = b_proj.reshape(1, D)
    idx, emb, loss = pl.pallas_call(
        _cra_kernel,
        grid=(B,),
        in_specs=[
            pl.BlockSpec((1, W, 2 * D), lambda i: (i, 0, 0)),
            pl.BlockSpec((1, K), lambda i: (0, 0)),
            pl.BlockSpec((K, D), lambda i: (0, 0)),
            pl.BlockSpec((D, D), lambda i: (0, 0)),
            pl.BlockSpec((1, D), lambda i: (0, 0)),
        ],
        out_specs=[
            pl.BlockSpec((1, 1, W), lambda i: (i, 0, 0)),
            pl.BlockSpec((1, W, D), lambda i: (i, 0, 0)),
            pl.BlockSpec(memory_space=pltpu.SMEM),
        ],
        out_shape=[
            jax.ShapeDtypeStruct((B, 1, W), jnp.int32),
            jax.ShapeDtypeStruct((B, W, D), jnp.float32),
            jax.ShapeDtypeStruct((1, 1), jnp.float32),
        ],
        scratch_shapes=[
            pltpu.VMEM((1, K), jnp.float32),
            pltpu.SMEM((1, 1), jnp.float32),
        ],
        compiler_params=pltpu.CompilerParams(
            dimension_semantics=("arbitrary",)),
    )(char_embeddings.reshape(B, W, 2 * D), c2.reshape(1, K),
      word_codebook, W_proj, b2)
    return (idx.reshape(B, W), emb, loss.reshape(()))


# speedup vs baseline: 2.1660x; 1.5638x over previous
import functools

import jax
import jax.numpy as jnp
from jax import lax
from jax.experimental import pallas as pl
from jax.experimental.pallas import tpu as pltpu
from jax.experimental.pallas import tpu_sc as plsc

B, L, D = 16, 1024, 256
W = L // 2
K = 2048
NB = B * W


def _rowsum_ref_order(sq):
    m = sq[:, :128] + sq[:, 128:]
    r = m[:, 0:8]
    for k in range(1, 16):
        r = r + m[:, 8 * k:8 * k + 8]
    u = r[:, 0:4] + r[:, 4:8]
    v = u[:, 0:2] + u[:, 2:4]
    return v[:, 0:1] + v[:, 1:2]


def _argmin_kernel(ce_ref, c2_ref, cb_ref, w_ref, b_ref,
                   idx_ref, s_ref):
    step = pl.program_id(0)

    @pl.when(step == 0)
    def _init():
        cb = cb_ref[...]
        proj = jax.lax.dot_general(
            cb, w_ref[...], (((1,), (0,)), ((), ())),
            precision=jax.lax.Precision.DEFAULT,
            preferred_element_type=jnp.float32) + b_ref[...]
        r = proj - cb
        s_ref[0, :] = jnp.sum(r * r, axis=1)

    ce = ce_ref[0]
    wm = (ce[:, :D] + ce[:, D:]) * 0.5

    cb = cb_ref[...]
    e = jax.lax.dot_general(
        wm, cb, (((1,), (1,)), ((), ())),
        precision=jax.lax.Precision.DEFAULT,
        preferred_element_type=jnp.float32)
    x2 = _rowsum_ref_order(wm * wm)
    d2 = (x2 - 2.0 * e) + c2_ref[0][None, :]

    lanes = jax.lax.broadcasted_iota(jnp.int32, (W, K), 1)
    mn = jnp.min(d2, axis=1, keepdims=True)
    idx = jnp.min(jnp.where(d2 == mn, lanes, K), axis=1).astype(jnp.int32)
    idx_ref[0, 0, :] = idx


def _tc_call(ce2, c2, cb, w, b2):
    return pl.pallas_call(
        _argmin_kernel,
        grid=(B,),
        in_specs=[
            pl.BlockSpec((1, W, 2 * D), lambda i: (i, 0, 0)),
            pl.BlockSpec((1, K), lambda i: (0, 0)),
            pl.BlockSpec((K, D), lambda i: (0, 0)),
            pl.BlockSpec((D, D), lambda i: (0, 0)),
            pl.BlockSpec((1, D), lambda i: (0, 0)),
        ],
        out_specs=[
            pl.BlockSpec((1, 1, W), lambda i: (i, 0, 0)),
            pl.BlockSpec((1, K), lambda i: (0, 0)),
        ],
        out_shape=[
            jax.ShapeDtypeStruct((B, 1, W), jnp.int32),
            jax.ShapeDtypeStruct((1, K), jnp.float32),
        ],
        compiler_params=pltpu.CompilerParams(
            dimension_semantics=("arbitrary",)),
    )(ce2, c2, cb, w, b2)


_NC, _NS = 2, 16
_NW = _NC * _NS
_RPW = NB // _NW
_CH = 128
_NCH = _RPW // _CH

_sc_cache = {}


def _get_sc_gather():
    if "k" in _sc_cache:
        return _sc_cache["k"]

    @functools.partial(
        pl.kernel,
        mesh=plsc.VectorSubcoreMesh(core_axis_name="c",
                                    subcore_axis_name="s"),
        out_type=[jax.ShapeDtypeStruct((NB, D), jnp.float32),
                  jax.ShapeDtypeStruct((_NW, 16), jnp.float32)],
        scratch_types=[pltpu.VMEM((_NCH, _CH), jnp.int32),
                       pltpu.VMEM((_RPW, D), jnp.float32),
                       pltpu.VMEM((_RPW,), jnp.float32),
                       pltpu.VMEM((16,), jnp.float32),
                       pltpu.SemaphoreType.DMA],
    )
    def _sc_gather(idx_hbm, cb_hbm, s_hbm, emb_hbm, part_hbm,
                   idx_v, rows_v, sv, pv, sem):
        wid = lax.axis_index("s") * _NC + lax.axis_index("c")
        base = wid * _RPW
        pltpu.sync_copy(idx_hbm.at[pl.ds(wid * _NCH, _NCH)], idx_v)
        copies = []
        for j in range(_NCH):
            copies.append(pltpu.async_copy(
                cb_hbm.at[idx_v.at[j]], rows_v.at[pl.ds(j * _CH, _CH)], sem))
        for j in range(_NCH):
            copies.append(pltpu.async_copy(
                s_hbm.at[idx_v.at[j]], sv.at[pl.ds(j * _CH, _CH)], sem))
        for c in copies:
            c.wait()
        pltpu.sync_copy(rows_v, emb_hbm.at[pl.ds(base, _RPW)])
        acc = sv[pl.ds(0, 16)]
        for i in range(1, _RPW // 16):
            acc = acc + sv[pl.ds(16 * i, 16)]
        pv[...] = acc
        pltpu.sync_copy(pv, part_hbm.at[wid])

    _sc_cache["k"] = _sc_gather
    return _sc_gather


@jax.jit
def kernel(char_tokens, char_embeddings, word_codebook, W_proj, b_proj):
    del char_tokens
    c2 = jnp.sum(word_codebook * word_codebook, axis=-1)
    b2 = b_proj.reshape(1, D)
    idx, s = _tc_call(char_embeddings.reshape(B, W, 2 * D), c2.reshape(1, K),
                      word_codebook, W_proj, b2)
    emb, parts = _get_sc_gather()(idx.reshape(NB // _CH, _CH), word_codebook,
                                  s.reshape(K))
    loss = jnp.sum(parts) * (1.0 / (NB * D))
    return (idx.reshape(B, W), emb.reshape(B, W, D), loss.reshape(()))


# speedup vs baseline: 2.1897x; 1.0110x over previous
import functools

import jax
import jax.numpy as jnp
from jax import lax
from jax.experimental import pallas as pl
from jax.experimental.pallas import tpu as pltpu
from jax.experimental.pallas import tpu_sc as plsc

B, L, D = 16, 1024, 256
W = L // 2
K = 2048
NB = B * W


def _rowsum_ref_order(sq):
    m = sq[:, :128] + sq[:, 128:]
    r = m[:, 0:8]
    for k in range(1, 16):
        r = r + m[:, 8 * k:8 * k + 8]
    u = r[:, 0:4] + r[:, 4:8]
    v = u[:, 0:2] + u[:, 2:4]
    return v[:, 0:1] + v[:, 1:2]


_BPS = 2
_M = _BPS * W
_NSTEP = B // _BPS


def _argmin_kernel(ce_ref, c2_ref, cb_ref, w_ref, b_ref,
                   idx_ref, s_ref):
    step = pl.program_id(0)

    @pl.when(step == 0)
    def _init():
        cb = cb_ref[...]
        proj = jax.lax.dot_general(
            cb, w_ref[...], (((1,), (0,)), ((), ())),
            precision=jax.lax.Precision.DEFAULT,
            preferred_element_type=jnp.float32) + b_ref[...]
        r = proj - cb
        s_ref[0, :] = jnp.sum(r * r, axis=1)

    ce = ce_ref[...].reshape(_M, 2 * D)
    wm = (ce[:, :D] + ce[:, D:]) * 0.5

    cb = cb_ref[...]
    e = jax.lax.dot_general(
        wm, cb, (((1,), (1,)), ((), ())),
        precision=jax.lax.Precision.DEFAULT,
        preferred_element_type=jnp.float32)
    x2 = _rowsum_ref_order(wm * wm)
    d2 = (x2 - 2.0 * e) + c2_ref[0][None, :]

    lanes = jax.lax.broadcasted_iota(jnp.int32, (_M, K), 1)
    mn = jnp.min(d2, axis=1, keepdims=True)
    idx = jnp.min(jnp.where(d2 == mn, lanes, K), axis=1).astype(jnp.int32)
    idx_ref[0, 0, :] = idx


def _tc_call(ce2, c2, cb, w, b2):
    return pl.pallas_call(
        _argmin_kernel,
        grid=(_NSTEP,),
        in_specs=[
            pl.BlockSpec((_BPS, W, 2 * D), lambda i: (i, 0, 0)),
            pl.BlockSpec((1, K), lambda i: (0, 0)),
            pl.BlockSpec((K, D), lambda i: (0, 0)),
            pl.BlockSpec((D, D), lambda i: (0, 0)),
            pl.BlockSpec((1, D), lambda i: (0, 0)),
        ],
        out_specs=[
            pl.BlockSpec((1, 1, _M), lambda i: (i, 0, 0)),
            pl.BlockSpec((1, K), lambda i: (0, 0)),
        ],
        out_shape=[
            jax.ShapeDtypeStruct((_NSTEP, 1, _M), jnp.int32),
            jax.ShapeDtypeStruct((1, K), jnp.float32),
        ],
        compiler_params=pltpu.CompilerParams(
            dimension_semantics=("arbitrary",)),
    )(ce2, c2, cb, w, b2)


_NC, _NS = 2, 16
_NW = _NC * _NS
_RPW = NB // _NW
_CH = 128
_NCH = _RPW // _CH

_sc_cache = {}


def _get_sc_gather():
    if "k" in _sc_cache:
        return _sc_cache["k"]

    @functools.partial(
        pl.kernel,
        mesh=plsc.VectorSubcoreMesh(core_axis_name="c",
                                    subcore_axis_name="s"),
        out_type=[jax.ShapeDtypeStruct((NB, D), jnp.float32),
                  jax.ShapeDtypeStruct((_NW, 16), jnp.float32)],
        scratch_types=[pltpu.VMEM((_NCH, _CH), jnp.int32),
                       pltpu.VMEM((_RPW, D), jnp.float32),
                       pltpu.VMEM((_RPW,), jnp.float32),
                       pltpu.VMEM((16,), jnp.float32),
                       pltpu.SemaphoreType.DMA],
    )
    def _sc_gather(idx_hbm, cb_hbm, s_hbm, emb_hbm, part_hbm,
                   idx_v, rows_v, sv, pv, sem):
        wid = lax.axis_index("s") * _NC + lax.axis_index("c")
        base = wid * _RPW
        pltpu.sync_copy(idx_hbm.at[pl.ds(wid * _NCH, _NCH)], idx_v)
        copies = []
        for j in range(_NCH):
            copies.append(pltpu.async_copy(
                cb_hbm.at[idx_v.at[j]], rows_v.at[pl.ds(j * _CH, _CH)], sem))
        for j in range(_NCH):
            copies.append(pltpu.async_copy(
                s_hbm.at[idx_v.at[j]], sv.at[pl.ds(j * _CH, _CH)], sem))
        for c in copies:
            c.wait()
        pltpu.sync_copy(rows_v, emb_hbm.at[pl.ds(base, _RPW)])
        acc = sv[pl.ds(0, 16)]
        for i in range(1, _RPW // 16):
            acc = acc + sv[pl.ds(16 * i, 16)]
        pv[...] = acc
        pltpu.sync_copy(pv, part_hbm.at[wid])

    _sc_cache["k"] = _sc_gather
    return _sc_gather


@jax.jit
def kernel(char_tokens, char_embeddings, word_codebook, W_proj, b_proj):
    del char_tokens
    c2 = jnp.sum(word_codebook * word_codebook, axis=-1)
    b2 = b_proj.reshape(1, D)
    idx, s = _tc_call(char_embeddings.reshape(B, W, 2 * D), c2.reshape(1, K),
                      word_codebook, W_proj, b2)
    emb, parts = _get_sc_gather()(idx.reshape(NB // _CH, _CH), word_codebook,
                                  s.reshape(K))
    loss = jnp.sum(parts) * (1.0 / (NB * D))
    return (idx.reshape(B, W), emb.reshape(B, W, D), loss.reshape(()))


# speedup vs baseline: 2.2521x; 1.0285x over previous
import functools

import jax
import jax.numpy as jnp
from jax import lax
from jax.experimental import pallas as pl
from jax.experimental.pallas import tpu as pltpu
from jax.experimental.pallas import tpu_sc as plsc

B, L, D = 16, 1024, 256
W = L // 2
K = 2048
NB = B * W


def _rowsum_ref_order(sq):
    m = sq[:, :128] + sq[:, 128:]
    r = m[:, 0:8]
    for k in range(1, 16):
        r = r + m[:, 8 * k:8 * k + 8]
    u = r[:, 0:4] + r[:, 4:8]
    v = u[:, 0:2] + u[:, 2:4]
    return v[:, 0:1] + v[:, 1:2]


_BPS = 2
_M = _BPS * W
_NSTEP = B // _BPS


def _argmin_kernel(ce_ref, c2_ref, cb_ref, w_ref, b_ref,
                   idx_ref, s_ref):
    step = pl.program_id(0)

    @pl.when(step == 0)
    def _init():
        cb = cb_ref[...]
        proj = jax.lax.dot_general(
            cb, w_ref[...], (((1,), (0,)), ((), ())),
            precision=jax.lax.Precision.DEFAULT,
            preferred_element_type=jnp.float32) + b_ref[...]
        r = proj - cb
        s_ref[0, :] = jnp.sum(r * r, axis=1)

    ce = ce_ref[...].reshape(_M, 2 * D)
    wm = (ce[:, :D] + ce[:, D:]) * 0.5

    cb = cb_ref[...]
    e = jax.lax.dot_general(
        wm, cb, (((1,), (1,)), ((), ())),
        precision=jax.lax.Precision.DEFAULT,
        preferred_element_type=jnp.float32)
    x2 = _rowsum_ref_order(wm * wm)
    d2 = (x2 - 2.0 * e) + c2_ref[0][None, :]

    lanes = jax.lax.broadcasted_iota(
        jnp.int32, (_M, K), 1).astype(jnp.float32)
    mn = jnp.min(d2, axis=1, keepdims=True)
    idxf = jnp.min(jnp.where(d2 == mn, lanes, float(K)), axis=1)
    idx_ref[0, 0, :] = idxf.astype(jnp.int32)


def _tc_call(ce2, c2, cb, w, b2):
    return pl.pallas_call(
        _argmin_kernel,
        grid=(_NSTEP,),
        in_specs=[
            pl.BlockSpec((_BPS, W, 2 * D), lambda i: (i, 0, 0)),
            pl.BlockSpec((1, K), lambda i: (0, 0)),
            pl.BlockSpec((K, D), lambda i: (0, 0)),
            pl.BlockSpec((D, D), lambda i: (0, 0)),
            pl.BlockSpec((1, D), lambda i: (0, 0)),
        ],
        out_specs=[
            pl.BlockSpec((1, 1, _M), lambda i: (i, 0, 0)),
            pl.BlockSpec((1, K), lambda i: (0, 0)),
        ],
        out_shape=[
            jax.ShapeDtypeStruct((_NSTEP, 1, _M), jnp.int32),
            jax.ShapeDtypeStruct((1, K), jnp.float32),
        ],
        compiler_params=pltpu.CompilerParams(
            dimension_semantics=("arbitrary",)),
    )(ce2, c2, cb, w, b2)


_NC, _NS = 2, 16
_NW = _NC * _NS
_RPW = NB // _NW
_CH = 128
_NCH = _RPW // _CH

_sc_cache = {}


def _get_sc_gather():
    if "k" in _sc_cache:
        return _sc_cache["k"]

    @functools.partial(
        pl.kernel,
        mesh=plsc.VectorSubcoreMesh(core_axis_name="c",
                                    subcore_axis_name="s"),
        out_type=[jax.ShapeDtypeStruct((NB, D), jnp.float32),
                  jax.ShapeDtypeStruct((_NW, 16), jnp.float32)],
        scratch_types=[pltpu.VMEM((_NCH, _CH), jnp.int32),
                       pltpu.VMEM((_RPW, D), jnp.float32),
                       pltpu.VMEM((_RPW,), jnp.float32),
                       pltpu.VMEM((16,), jnp.float32),
                       pltpu.SemaphoreType.DMA],
    )
    def _sc_gather(idx_hbm, cb_hbm, s_hbm, emb_hbm, part_hbm,
                   idx_v, rows_v, sv, pv, sem):
        wid = lax.axis_index("s") * _NC + lax.axis_index("c")
        base = wid * _RPW
        pltpu.sync_copy(idx_hbm.at[pl.ds(wid * _NCH, _NCH)], idx_v)
        copies = []
        for j in range(_NCH):
            copies.append(pltpu.async_copy(
                cb_hbm.at[idx_v.at[j]], rows_v.at[pl.ds(j * _CH, _CH)], sem))
        for j in range(_NCH):
            copies.append(pltpu.async_copy(
                s_hbm.at[idx_v.at[j]], sv.at[pl.ds(j * _CH, _CH)], sem))
        for c in copies:
            c.wait()
        pltpu.sync_copy(rows_v, emb_hbm.at[pl.ds(base, _RPW)])
        acc = sv[pl.ds(0, 16)]
        for i in range(1, _RPW // 16):
            acc = acc + sv[pl.ds(16 * i, 16)]
        pv[...] = acc
        pltpu.sync_copy(pv, part_hbm.at[wid])

    _sc_cache["k"] = _sc_gather
    return _sc_gather


@jax.jit
def kernel(char_tokens, char_embeddings, word_codebook, W_proj, b_proj):
    del char_tokens
    c2 = jnp.sum(word_codebook * word_codebook, axis=-1)
    b2 = b_proj.reshape(1, D)
    idx, s = _tc_call(char_embeddings.reshape(B, W, 2 * D), c2.reshape(1, K),
                      word_codebook, W_proj, b2)
    emb, parts = _get_sc_gather()(idx.reshape(NB // _CH, _CH), word_codebook,
                                  s.reshape(K))
    loss = jnp.sum(parts) * (1.0 / (NB * D))
    return (idx.reshape(B, W), emb.reshape(B, W, D), loss.reshape(()))
